# Initial kernel scaffold; baseline (speedup 1.0000x reference)
import jax
import jax.numpy as jnp
from jax.experimental import pallas as pl


def kernel(x, edge_index, W1, b1, W2, b2):
    raise NotImplementedError("write your pallas kernel here")


# speedup vs baseline: 3.4356x; 3.4356x over previous
import jax
import jax.numpy as jnp
from jax import lax
from jax.experimental import pallas as pl
from jax.experimental.pallas import tpu as pltpu
from jax.experimental.pallas import tpu_sc as plsc

NC = 2
NS = 16
NW = NC * NS
EB = 128
F = 128


def _lin1_body(x_ref, w_ref, o_ref):
    o_ref[...] = jnp.dot(x_ref[...], w_ref[...],
                         preferred_element_type=jnp.float32)


def _lin1(x, w1t):
    n = x.shape[0]
    blk = 2000
    return pl.pallas_call(
        _lin1_body,
        grid=(n // blk,),
        in_specs=[
            pl.BlockSpec((blk, F), lambda i: (i, 0)),
            pl.BlockSpec((F, F), lambda i: (0, 0)),
        ],
        out_specs=pl.BlockSpec((blk, F), lambda i: (i, 0)),
        out_shape=jax.ShapeDtypeStruct((n, F), jnp.float32),
    )(x, w1t)


CH = 8


def _make_agg(npad, nb):
    rows_per_tile = npad // NS
    nz = rows_per_tile // EB
    nch = nb // CH

    def body(g_hbm, src_hbm, dst_hbm, outp_hbm, outd_hbm,
             acc_sh, deg_sh, src_c, dst_c, rows0, rows1, ones_v, zdeg_v,
             sem0, sem1):
        c = lax.axis_index("c")
        s = lax.axis_index("s")
        wid = s * NC + c

        z16 = jnp.zeros((16,), jnp.float32)
        one16 = jnp.full((16,), 1.0, jnp.float32)

        def zrow(r, carry):
            for lg in range(F // 16):
                rows0[r, pl.ds(lg * 16, 16)] = z16
            return carry

        lax.fori_loop(0, EB, zrow, 0)

        for i in range(EB // 16):
            ones_v[pl.ds(i * 16, 16)] = one16
        for i in range(rows_per_tile // 16):
            zdeg_v[pl.ds(i * 16, 16)] = z16

        row0 = s * rows_per_tile
        for b in range(nz):
            pltpu.sync_copy(rows0, acc_sh.at[pl.ds(row0 + b * EB, EB)])
        pltpu.sync_copy(zdeg_v, deg_sh.at[pl.ds(row0, rows_per_tile)])

        plsc.subcore_barrier()

        def chunk(k, carry):
            pltpu.sync_copy(src_hbm.at[wid].at[pl.ds(k * CH, CH)], src_c)
            pltpu.sync_copy(dst_hbm.at[wid].at[pl.ds(k * CH, CH)], dst_c)
            for j in range(0, CH, 2):
                cp0 = pltpu.make_async_copy(
                    g_hbm.at[dst_c.at[j]], rows0, sem0)
                cp0.start()
                cp1 = pltpu.make_async_copy(
                    g_hbm.at[dst_c.at[j + 1]], rows1, sem1)
                cp1.start()
                cp0.wait()
                pltpu.sync_copy(rows0, acc_sh.at[src_c.at[j]], add=True)
                pltpu.sync_copy(ones_v, deg_sh.at[src_c.at[j]], add=True)
                cp1.wait()
                pltpu.sync_copy(rows1, acc_sh.at[src_c.at[j + 1]], add=True)
                pltpu.sync_copy(ones_v, deg_sh.at[src_c.at[j + 1]], add=True)
            return carry

        lax.fori_loop(0, nch, chunk, 0)
        plsc.subcore_barrier()

        pltpu.sync_copy(acc_sh.at[pl.ds(row0, rows_per_tile)],
                        outp_hbm.at[c, pl.ds(row0, rows_per_tile)])
        pltpu.sync_copy(deg_sh.at[pl.ds(row0, rows_per_tile)],
                        outd_hbm.at[c, pl.ds(row0, rows_per_tile)])

    return pl.kernel(
        body,
        mesh=plsc.VectorSubcoreMesh(core_axis_name="c", subcore_axis_name="s"),
        out_type=(
            jax.ShapeDtypeStruct((NC, npad, F), jnp.float32),
            jax.ShapeDtypeStruct((NC, npad), jnp.float32),
        ),
        scratch_types=(
            pltpu.VMEM_SHARED((npad, F), jnp.float32),
            pltpu.VMEM_SHARED((npad,), jnp.float32),
            pltpu.VMEM((CH, EB), jnp.int32),
            pltpu.VMEM((CH, EB), jnp.int32),
            pltpu.VMEM((EB, F), jnp.float32),
            pltpu.VMEM((EB, F), jnp.float32),
            pltpu.VMEM((EB,), jnp.float32),
            pltpu.VMEM((rows_per_tile,), jnp.float32),
            pltpu.SemaphoreType.DMA,
            pltpu.SemaphoreType.DMA,
        ),
    )


def _mid_body(p0, p1, d0, d1, b_ref, w_ref, o_ref):
    deg = d0[...] + d1[...]
    m1n = (p0[...] + p1[...]) * (1.0 / deg)
    h = jnp.maximum(m1n + b_ref[...], 0.0)
    o_ref[...] = jnp.dot(h, w_ref[...], preferred_element_type=jnp.float32)


def _mid(p0, p1, d0, d1, b1r, w2t):
    n = p0.shape[0]
    blk = 2000
    return pl.pallas_call(
        _mid_body,
        grid=(n // blk,),
        in_specs=[
            pl.BlockSpec((blk, F), lambda i: (i, 0)),
            pl.BlockSpec((blk, F), lambda i: (i, 0)),
            pl.BlockSpec((blk, 1), lambda i: (i, 0)),
            pl.BlockSpec((blk, 1), lambda i: (i, 0)),
            pl.BlockSpec((1, F), lambda i: (0, 0)),
            pl.BlockSpec((F, F), lambda i: (0, 0)),
        ],
        out_specs=pl.BlockSpec((blk, F), lambda i: (i, 0)),
        out_shape=jax.ShapeDtypeStruct((n, F), jnp.float32),
    )(p0, p1, d0, d1, b1r, w2t)


def _out_body(q0, q1, d0, d1, b_ref, o_ref):
    ncls = o_ref.shape[1]
    deg = d0[...] + d1[...]
    z = (q0[...] + q1[...]) * (1.0 / deg) + b_ref[...]
    lane = lax.broadcasted_iota(jnp.int32, z.shape, 1)
    valid = lane < ncls
    zm = jnp.where(valid, z, -jnp.inf)
    m = jnp.max(zm, axis=1, keepdims=True)
    e = jnp.where(valid, jnp.exp(z - m), 0.0)
    sz = jnp.sum(e, axis=1, keepdims=True)
    res = z - m - jnp.log(sz)
    o_ref[...] = res[:, :ncls]


def _out(q0, q1, d0, d1, b2r, ncls):
    n = q0.shape[0]
    blk = 2000
    return pl.pallas_call(
        _out_body,
        grid=(n // blk,),
        in_specs=[
            pl.BlockSpec((blk, F), lambda i: (i, 0)),
            pl.BlockSpec((blk, F), lambda i: (i, 0)),
            pl.BlockSpec((blk, 1), lambda i: (i, 0)),
            pl.BlockSpec((blk, 1), lambda i: (i, 0)),
            pl.BlockSpec((1, F), lambda i: (0, 0)),
        ],
        out_specs=pl.BlockSpec((blk, ncls), lambda i: (i, 0)),
        out_shape=jax.ShapeDtypeStruct((n, ncls), jnp.float32),
    )(q0, q1, d0, d1, b2r)


def kernel(x, edge_index, W1, b1, W2, b2):
    n = x.shape[0]
    e = edge_index.shape[1]
    ncls = W2.shape[0]

    npad = ((n + NS * EB) // (NS * EB)) * (NS * EB)

    ept = ((e + NW * 2 * EB - 1) // (NW * 2 * EB)) * (2 * EB)
    nb = ept // EB
    pad = NW * ept - e
    src3 = jnp.concatenate(
        [edge_index[0], jnp.full((pad,), n, jnp.int32)]).reshape(NW, nb, EB)
    dst3 = jnp.concatenate(
        [edge_index[1], jnp.zeros((pad,), jnp.int32)]).reshape(NW, nb, EB)

    w1t = W1.T
    b1r = b1.reshape(1, F)
    w2t = jnp.zeros((F, F), jnp.float32).at[:, :ncls].set(W2.T)
    b2r = jnp.zeros((1, F), jnp.float32).at[:, :ncls].set(b2.reshape(1, ncls))

    g1 = _lin1(x, w1t)

    agg = _make_agg(npad, nb)
    acc1, degp = agg(g1, src3, dst3)
    d0 = degp[0, :n, None]
    d1 = degp[1, :n, None]

    g2 = _mid(acc1[0, :n], acc1[1, :n], d0, d1, b1r, w2t)

    acc2, _ = agg(g2, src3, dst3)

    return _out(acc2[0, :n], acc2[1, :n], d0, d1, b2r, ncls)


# speedup vs baseline: 3.8369x; 1.1168x over previous
import jax
import jax.numpy as jnp
from jax import lax
from jax.experimental import pallas as pl
from jax.experimental.pallas import tpu as pltpu
from jax.experimental.pallas import tpu_sc as plsc

NC = 2
NS = 16
NW = NC * NS
EB = 128
F = 128


def _lin1_body(x_ref, w_ref, o_ref):
    o_ref[...] = jnp.dot(x_ref[...], w_ref[...],
                         preferred_element_type=jnp.float32)


def _lin1(x, w1t):
    n = x.shape[0]
    blk = 2000
    return pl.pallas_call(
        _lin1_body,
        grid=(n // blk,),
        in_specs=[
            pl.BlockSpec((blk, F), lambda i: (i, 0)),
            pl.BlockSpec((F, F), lambda i: (0, 0)),
        ],
        out_specs=pl.BlockSpec((blk, F), lambda i: (i, 0)),
        out_shape=jax.ShapeDtypeStruct((n, F), jnp.float32),
    )(x, w1t)


CH = 8


def _make_agg(npad, nb):
    rows_per_tile = npad // NS
    nz = rows_per_tile // EB
    nch = nb // CH

    def body(g_hbm, src_hbm, dst_hbm, outp_hbm, outd_hbm,
             acc_sh, deg_sh, src_c, dst_c, rows0, rows1, ones_v, zdeg_v,
             sem0, sem1):
        c = lax.axis_index("c")
        s = lax.axis_index("s")
        wid = s * NC + c

        z16 = jnp.zeros((16,), jnp.float32)
        one16 = jnp.full((16,), 1.0, jnp.float32)

        def zrow(r, carry):
            for lg in range(F // 16):
                rows0[r, pl.ds(lg * 16, 16)] = z16
            return carry

        lax.fori_loop(0, EB, zrow, 0)

        for i in range(EB // 16):
            ones_v[pl.ds(i * 16, 16)] = one16
        for i in range(rows_per_tile // 16):
            zdeg_v[pl.ds(i * 16, 16)] = z16

        row0 = s * rows_per_tile
        for b in range(nz):
            pltpu.sync_copy(rows0, acc_sh.at[pl.ds(row0 + b * EB, EB)])
        pltpu.sync_copy(zdeg_v, deg_sh.at[pl.ds(row0, rows_per_tile)])

        plsc.subcore_barrier()

        def chunk(k, carry):
            pltpu.sync_copy(src_hbm.at[wid].at[pl.ds(k * CH, CH)], src_c)
            pltpu.sync_copy(dst_hbm.at[wid].at[pl.ds(k * CH, CH)], dst_c)
            for j in range(0, CH, 2):
                cp0 = pltpu.make_async_copy(
                    g_hbm.at[dst_c.at[j]], rows0, sem0)
                cp0.start()
                cp1 = pltpu.make_async_copy(
                    g_hbm.at[dst_c.at[j + 1]], rows1, sem1)
                cp1.start()
                cp0.wait()
                pltpu.sync_copy(rows0, acc_sh.at[src_c.at[j]], add=True)
                pltpu.sync_copy(ones_v, deg_sh.at[src_c.at[j]], add=True)
                cp1.wait()
                pltpu.sync_copy(rows1, acc_sh.at[src_c.at[j + 1]], add=True)
                pltpu.sync_copy(ones_v, deg_sh.at[src_c.at[j + 1]], add=True)
            return carry

        lax.fori_loop(0, nch, chunk, 0)
        plsc.subcore_barrier()

        pltpu.sync_copy(acc_sh.at[pl.ds(row0, rows_per_tile)],
                        outp_hbm.at[c, pl.ds(row0, rows_per_tile)])
        pltpu.sync_copy(deg_sh.at[pl.ds(row0, rows_per_tile)],
                        outd_hbm.at[c, pl.ds(row0, rows_per_tile)])

    return pl.kernel(
        body,
        mesh=plsc.VectorSubcoreMesh(core_axis_name="c", subcore_axis_name="s"),
        out_type=(
            jax.ShapeDtypeStruct((NC, npad, F), jnp.float32),
            jax.ShapeDtypeStruct((NC, npad), jnp.float32),
        ),
        scratch_types=(
            pltpu.VMEM_SHARED((npad, F), jnp.float32),
            pltpu.VMEM_SHARED((npad,), jnp.float32),
            pltpu.VMEM((CH, EB), jnp.int32),
            pltpu.VMEM((CH, EB), jnp.int32),
            pltpu.VMEM((EB, F), jnp.float32),
            pltpu.VMEM((EB, F), jnp.float32),
            pltpu.VMEM((EB,), jnp.float32),
            pltpu.VMEM((rows_per_tile,), jnp.float32),
            pltpu.SemaphoreType.DMA,
            pltpu.SemaphoreType.DMA,
        ),
    )


def _mid_body(p0, p1, d0, d1, b_ref, w_ref, o_ref):
    deg = d0[...] + d1[...]
    m1n = (p0[...] + p1[...]) * (1.0 / deg)
    h = jnp.maximum(m1n + b_ref[...], 0.0)
    o_ref[...] = jnp.dot(h, w_ref[...], preferred_element_type=jnp.float32)


def _mid(p0, p1, d0, d1, b1r, w2t):
    n = p0.shape[0]
    blk = 2000
    return pl.pallas_call(
        _mid_body,
        grid=(n // blk,),
        in_specs=[
            pl.BlockSpec((blk, F), lambda i: (i, 0)),
            pl.BlockSpec((blk, F), lambda i: (i, 0)),
            pl.BlockSpec((blk, 1), lambda i: (i, 0)),
            pl.BlockSpec((blk, 1), lambda i: (i, 0)),
            pl.BlockSpec((1, F), lambda i: (0, 0)),
            pl.BlockSpec((F, F), lambda i: (0, 0)),
        ],
        out_specs=pl.BlockSpec((blk, F), lambda i: (i, 0)),
        out_shape=jax.ShapeDtypeStruct((n, F), jnp.float32),
    )(p0, p1, d0, d1, b1r, w2t)


def _out_body(q0, q1, d0, d1, b_ref, o_ref):
    ncls = o_ref.shape[1]
    deg = d0[...] + d1[...]
    z = (q0[...] + q1[...]) * (1.0 / deg) + b_ref[...]
    lane = lax.broadcasted_iota(jnp.int32, z.shape, 1)
    valid = lane < ncls
    zm = jnp.where(valid, z, -jnp.inf)
    m = jnp.max(zm, axis=1, keepdims=True)
    e = jnp.where(valid, jnp.exp(z - m), 0.0)
    sz = jnp.sum(e, axis=1, keepdims=True)
    res = z - m - jnp.log(sz)
    o_ref[...] = res[:, :ncls]


def _out(q0, q1, d0, d1, b2r, ncls):
    n = q0.shape[0]
    blk = 2000
    return pl.pallas_call(
        _out_body,
        grid=(n // blk,),
        in_specs=[
            pl.BlockSpec((blk, F), lambda i: (i, 0)),
            pl.BlockSpec((blk, F), lambda i: (i, 0)),
            pl.BlockSpec((blk, 1), lambda i: (i, 0)),
            pl.BlockSpec((blk, 1), lambda i: (i, 0)),
            pl.BlockSpec((1, F), lambda i: (0, 0)),
        ],
        out_specs=pl.BlockSpec((blk, ncls), lambda i: (i, 0)),
        out_shape=jax.ShapeDtypeStruct((n, ncls), jnp.float32),
    )(q0, q1, d0, d1, b2r)


def kernel(x, edge_index, W1, b1, W2, b2):
    n = x.shape[0]
    e = edge_index.shape[1]
    ncls = W2.shape[0]

    npad = ((n + NS * EB) // (NS * EB)) * (NS * EB)

    epw = -(-e // NW)
    nb = (-(-(-(-epw // EB)) // CH)) * CH
    slot = nb * EB
    flat_pad = NW * epw - e
    spare = npad - n
    padcols = slot - epw
    srcw = jnp.concatenate(
        [edge_index[0], jnp.full((flat_pad,), n, jnp.int32)]).reshape(NW, epw)
    dstw = jnp.concatenate(
        [edge_index[1], jnp.zeros((flat_pad,), jnp.int32)]).reshape(NW, epw)
    dump = jnp.broadcast_to(
        n + (jnp.arange(padcols, dtype=jnp.int32) % spare), (NW, padcols))
    zpad = jnp.zeros((NW, padcols), jnp.int32)
    src3 = jnp.concatenate([srcw, dump], axis=1).reshape(NW, nb, EB)
    dst3 = jnp.concatenate([dstw, zpad], axis=1).reshape(NW, nb, EB)

    w1t = W1.T
    b1r = b1.reshape(1, F)
    w2t = jnp.zeros((F, F), jnp.float32).at[:, :ncls].set(W2.T)
    b2r = jnp.zeros((1, F), jnp.float32).at[:, :ncls].set(b2.reshape(1, ncls))

    g1 = _lin1(x, w1t)

    agg = _make_agg(npad, nb)
    acc1, degp = agg(g1, src3, dst3)
    d0 = degp[0, :n, None]
    d1 = degp[1, :n, None]

    g2 = _mid(acc1[0, :n], acc1[1, :n], d0, d1, b1r, w2t)

    acc2, _ = agg(g2, src3, dst3)

    return _out(acc2[0, :n], acc2[1, :n], d0, d1, b2r, ncls)


# speedup vs baseline: 4.1373x; 1.0783x over previous
import jax
import jax.numpy as jnp
from jax import lax
from jax.experimental import pallas as pl
from jax.experimental.pallas import tpu as pltpu
from jax.experimental.pallas import tpu_sc as plsc

NC = 2
NS = 16
NW = NC * NS
EB = 128
F = 128


def _lin1_body(x_ref, w_ref, o_ref):
    o_ref[...] = jnp.dot(x_ref[...], w_ref[...],
                         preferred_element_type=jnp.float32)


def _lin1(x, w1t):
    n = x.shape[0]
    blk = 2000
    return pl.pallas_call(
        _lin1_body,
        grid=(n // blk,),
        in_specs=[
            pl.BlockSpec((blk, F), lambda i: (i, 0)),
            pl.BlockSpec((F, F), lambda i: (0, 0)),
        ],
        out_specs=pl.BlockSpec((blk, F), lambda i: (i, 0)),
        out_shape=jax.ShapeDtypeStruct((n, F), jnp.float32),
    )(x, w1t)


CH = 8
BPB = 16
NBUF = 2


def _make_agg(npad, nb):
    rows_per_tile = npad // NS
    nz = rows_per_tile // EB
    nch = nb // CH
    nbod = nch // 2

    def body(g_hbm, src_hbm, dst_hbm, outp_hbm, outd_hbm,
             acc_sh, deg_sh, src_a, dst_a, src_b, dst_b,
             r0, r1, ones_v, zdeg_v,
             sg0, sg1, ss0, ss1, sd, si0, si1):
        c = lax.axis_index("c")
        s = lax.axis_index("s")
        wid = s * NC + c
        rows = (r0, r1)
        gsem = (sg0, sg1)
        ssem = (ss0, ss1)
        isrc = (src_a, src_b)
        idst = (dst_a, dst_b)
        isem = (si0, si1)

        z16 = jnp.zeros((16,), jnp.float32)
        one16 = jnp.full((16,), 1.0, jnp.float32)

        def zrow(r, carry):
            for lg in range(F // 16):
                r0[r, pl.ds(lg * 16, 16)] = z16
            return carry

        lax.fori_loop(0, EB, zrow, 0)
        for i in range(EB // 16):
            ones_v[pl.ds(i * 16, 16)] = one16
        for i in range(rows_per_tile // 16):
            zdeg_v[pl.ds(i * 16, 16)] = z16

        row0 = s * rows_per_tile
        for b in range(nz):
            pltpu.sync_copy(r0, acc_sh.at[pl.ds(row0 + b * EB, EB)])
        pltpu.sync_copy(zdeg_v, deg_sh.at[pl.ds(row0, rows_per_tile)])

        def g_desc(t):
            buf = idst[0] if t < CH else idst[1]
            return pltpu.make_async_copy(
                g_hbm.at[buf.at[t % CH]], rows[t % NBUF], gsem[t % NBUF])

        def s_desc(t):
            buf = isrc[0] if t < CH else isrc[1]
            return pltpu.make_async_copy(
                rows[t % NBUF], acc_sh.at[buf.at[t % CH]], ssem[t % NBUF])

        def d_desc(t):
            buf = isrc[0] if t < CH else isrc[1]
            return pltpu.make_async_copy(
                ones_v, deg_sh.at[buf.at[t % CH]], sd)

        def idx_start(k, p):
            pltpu.make_async_copy(
                src_hbm.at[wid].at[pl.ds(k * CH, CH)], isrc[p],
                isem[p]).start()
            pltpu.make_async_copy(
                dst_hbm.at[wid].at[pl.ds(k * CH, CH)], idst[p],
                isem[p]).start()

        def idx_wait(p):
            pltpu.make_async_copy(
                src_hbm.at[wid].at[pl.ds(0, CH)], isrc[p], isem[p]).wait()
            pltpu.make_async_copy(
                dst_hbm.at[wid].at[pl.ds(0, CH)], idst[p], isem[p]).wait()

        idx_start(0, 0)
        idx_start(1, 1)
        idx_wait(0)
        g_desc(0).start()

        plsc.subcore_barrier()

        def fbody(i, carry):
            for t in range(BPB):
                g_desc(t).wait()
                s_desc(t).start(add=True)
                if t < 2:
                    @pl.when(i > 0)
                    def _():
                        d_desc(t).wait()
                else:
                    d_desc(t - 2).wait()
                d_desc(t).start(add=True)
                if t == 0:
                    @pl.when(i > 0)
                    def _():
                        s_desc(BPB - 1).wait()
                    g_desc(1).start()
                else:
                    s_desc(t - 1).wait()
                    if t == 2:
                        @pl.when(i > 0)
                        def _():
                            idx_start(2 * i + 1, 1)
                    if t == 6:
                        idx_wait(1)
                    if t == 9:
                        @pl.when(i < nbod - 1)
                        def _():
                            idx_start(2 * i + 2, 0)
                    if t == 12:
                        @pl.when(i < nbod - 1)
                        def _():
                            idx_wait(0)
                    if t < BPB - 1:
                        g_desc(t + 1).start()
                    else:
                        @pl.when(i < nbod - 1)
                        def _():
                            g_desc(t + 1 - BPB).start()
            return carry

        lax.fori_loop(0, nbod, fbody, 0)

        s_desc(BPB - 1).wait()
        for t in range(2):
            d_desc(t).wait()

        plsc.subcore_barrier()

        pltpu.sync_copy(acc_sh.at[pl.ds(row0, rows_per_tile)],
                        outp_hbm.at[c, pl.ds(row0, rows_per_tile)])
        pltpu.sync_copy(deg_sh.at[pl.ds(row0, rows_per_tile)],
                        outd_hbm.at[c, pl.ds(row0, rows_per_tile)])

    return pl.kernel(
        body,
        mesh=plsc.VectorSubcoreMesh(core_axis_name="c", subcore_axis_name="s"),
        out_type=(
            jax.ShapeDtypeStruct((NC, npad, F), jnp.float32),
            jax.ShapeDtypeStruct((NC, npad), jnp.float32),
        ),
        scratch_types=(
            pltpu.VMEM_SHARED((npad, F), jnp.float32),
            pltpu.VMEM_SHARED((npad,), jnp.float32),
            pltpu.VMEM((CH, EB), jnp.int32),
            pltpu.VMEM((CH, EB), jnp.int32),
            pltpu.VMEM((CH, EB), jnp.int32),
            pltpu.VMEM((CH, EB), jnp.int32),
            pltpu.VMEM((EB, F), jnp.float32),
            pltpu.VMEM((EB, F), jnp.float32),
            pltpu.VMEM((EB,), jnp.float32),
            pltpu.VMEM((rows_per_tile,), jnp.float32),
            pltpu.SemaphoreType.DMA,
            pltpu.SemaphoreType.DMA,
            pltpu.SemaphoreType.DMA,
            pltpu.SemaphoreType.DMA,
            pltpu.SemaphoreType.DMA,
            pltpu.SemaphoreType.DMA,
            pltpu.SemaphoreType.DMA,
        ),
    )


def _mid_body(p0, p1, d0, d1, b_ref, w_ref, o_ref):
    deg = d0[...] + d1[...]
    m1n = (p0[...] + p1[...]) * (1.0 / deg)
    h = jnp.maximum(m1n + b_ref[...], 0.0)
    o_ref[...] = jnp.dot(h, w_ref[...], preferred_element_type=jnp.float32)


def _mid(p0, p1, d0, d1, b1r, w2t):
    n = p0.shape[0]
    blk = 2000
    return pl.pallas_call(
        _mid_body,
        grid=(n // blk,),
        in_specs=[
            pl.BlockSpec((blk, F), lambda i: (i, 0)),
            pl.BlockSpec((blk, F), lambda i: (i, 0)),
            pl.BlockSpec((blk, 1), lambda i: (i, 0)),
            pl.BlockSpec((blk, 1), lambda i: (i, 0)),
            pl.BlockSpec((1, F), lambda i: (0, 0)),
            pl.BlockSpec((F, F), lambda i: (0, 0)),
        ],
        out_specs=pl.BlockSpec((blk, F), lambda i: (i, 0)),
        out_shape=jax.ShapeDtypeStruct((n, F), jnp.float32),
    )(p0, p1, d0, d1, b1r, w2t)


def _out_body(q0, q1, d0, d1, b_ref, o_ref):
    ncls = o_ref.shape[1]
    deg = d0[...] + d1[...]
    z = (q0[...] + q1[...]) * (1.0 / deg) + b_ref[...]
    lane = lax.broadcasted_iota(jnp.int32, z.shape, 1)
    valid = lane < ncls
    zm = jnp.where(valid, z, -jnp.inf)
    m = jnp.max(zm, axis=1, keepdims=True)
    e = jnp.where(valid, jnp.exp(z - m), 0.0)
    sz = jnp.sum(e, axis=1, keepdims=True)
    res = z - m - jnp.log(sz)
    o_ref[...] = res[:, :ncls]


def _out(q0, q1, d0, d1, b2r, ncls):
    n = q0.shape[0]
    blk = 2000
    return pl.pallas_call(
        _out_body,
        grid=(n // blk,),
        in_specs=[
            pl.BlockSpec((blk, F), lambda i: (i, 0)),
            pl.BlockSpec((blk, F), lambda i: (i, 0)),
            pl.BlockSpec((blk, 1), lambda i: (i, 0)),
            pl.BlockSpec((blk, 1), lambda i: (i, 0)),
            pl.BlockSpec((1, F), lambda i: (0, 0)),
        ],
        out_specs=pl.BlockSpec((blk, ncls), lambda i: (i, 0)),
        out_shape=jax.ShapeDtypeStruct((n, ncls), jnp.float32),
    )(q0, q1, d0, d1, b2r)


def kernel(x, edge_index, W1, b1, W2, b2):
    n = x.shape[0]
    e = edge_index.shape[1]
    ncls = W2.shape[0]

    npad = ((n + NS * EB) // (NS * EB)) * (NS * EB)

    epw = -(-e // NW)
    nb = (-(-(-(-epw // EB)) // CH)) * CH
    slot = nb * EB
    flat_pad = NW * epw - e
    spare = npad - n
    padcols = slot - epw
    srcw = jnp.concatenate(
        [edge_index[0], jnp.full((flat_pad,), n, jnp.int32)]).reshape(NW, epw)
    dstw = jnp.concatenate(
        [edge_index[1], jnp.zeros((flat_pad,), jnp.int32)]).reshape(NW, epw)
    dump = jnp.broadcast_to(
        n + (jnp.arange(padcols, dtype=jnp.int32) % spare), (NW, padcols))
    zpad = jnp.zeros((NW, padcols), jnp.int32)
    src3 = jnp.concatenate([srcw, dump], axis=1).reshape(NW, nb, EB)
    dst3 = jnp.concatenate([dstw, zpad], axis=1).reshape(NW, nb, EB)

    w1t = W1.T
    b1r = b1.reshape(1, F)
    w2t = jnp.zeros((F, F), jnp.float32).at[:, :ncls].set(W2.T)
    b2r = jnp.zeros((1, F), jnp.float32).at[:, :ncls].set(b2.reshape(1, ncls))

    g1 = _lin1(x, w1t)

    agg = _make_agg(npad, nb)
    acc1, degp = agg(g1, src3, dst3)
    d0 = degp[0, :n, None]
    d1 = degp[1, :n, None]

    g2 = _mid(acc1[0, :n], acc1[1, :n], d0, d1, b1r, w2t)

    acc2, _ = agg(g2, src3, dst3)

    return _out(acc2[0, :n], acc2[1, :n], d0, d1, b2r, ncls)


# speedup vs baseline: 4.1521x; 1.0036x over previous
import jax
import jax.numpy as jnp
from jax import lax
from jax.experimental import pallas as pl
from jax.experimental.pallas import tpu as pltpu
from jax.experimental.pallas import tpu_sc as plsc

NC = 2
NS = 16
NW = NC * NS
EB = 128
F = 128


def _lin1_body(x_ref, w_ref, o_ref):
    o_ref[...] = jnp.dot(x_ref[...], w_ref[...],
                         preferred_element_type=jnp.float32)


def _lin1(x, w1t):
    n = x.shape[0]
    blk = 2000
    return pl.pallas_call(
        _lin1_body,
        grid=(n // blk,),
        in_specs=[
            pl.BlockSpec((blk, F), lambda i: (i, 0)),
            pl.BlockSpec((F, F), lambda i: (0, 0)),
        ],
        out_specs=pl.BlockSpec((blk, F), lambda i: (i, 0)),
        out_shape=jax.ShapeDtypeStruct((n, F), jnp.float32),
    )(x, w1t)


CH = 8
BPB = 16
NBUF = 2


def _make_agg(npad, nb, with_deg):
    rows_per_tile = npad // NS
    nz = rows_per_tile // EB
    nch = nb // CH
    nbod = nch // 2

    def body(g_hbm, src_hbm, dst_hbm, outp_hbm, outd_hbm,
             acc_sh, deg_sh, src_a, dst_a, src_b, dst_b,
             r0, r1, ones_v, zdeg_v,
             sg0, sg1, ss0, ss1, sd, si0, si1):
        c = lax.axis_index("c")
        s = lax.axis_index("s")
        wid = s * NC + c
        rows = (r0, r1)
        gsem = (sg0, sg1)
        ssem = (ss0, ss1)
        isrc = (src_a, src_b)
        idst = (dst_a, dst_b)
        isem = (si0, si1)

        z16 = jnp.zeros((16,), jnp.float32)
        one16 = jnp.full((16,), 1.0, jnp.float32)

        def zrow(r, carry):
            for lg in range(F // 16):
                r0[r, pl.ds(lg * 16, 16)] = z16
            return carry

        lax.fori_loop(0, EB, zrow, 0)
        if with_deg:
            for i in range(EB // 16):
                ones_v[pl.ds(i * 16, 16)] = one16
            for i in range(rows_per_tile // 16):
                zdeg_v[pl.ds(i * 16, 16)] = z16

        row0 = s * rows_per_tile
        for b in range(nz):
            pltpu.sync_copy(r0, acc_sh.at[pl.ds(row0 + b * EB, EB)])
        if with_deg:
            pltpu.sync_copy(zdeg_v, deg_sh.at[pl.ds(row0, rows_per_tile)])

        def g_desc(t):
            buf = idst[0] if t < CH else idst[1]
            return pltpu.make_async_copy(
                g_hbm.at[buf.at[t % CH]], rows[t % NBUF], gsem[t % NBUF])

        def s_desc(t):
            buf = isrc[0] if t < CH else isrc[1]
            return pltpu.make_async_copy(
                rows[t % NBUF], acc_sh.at[buf.at[t % CH]], ssem[t % NBUF])

        def d_desc(t):
            buf = isrc[0] if t < CH else isrc[1]
            return pltpu.make_async_copy(
                ones_v, deg_sh.at[buf.at[t % CH]], sd)

        def idx_start(k, p):
            pltpu.make_async_copy(
                src_hbm.at[wid].at[pl.ds(k * CH, CH)], isrc[p],
                isem[p]).start()
            pltpu.make_async_copy(
                dst_hbm.at[wid].at[pl.ds(k * CH, CH)], idst[p],
                isem[p]).start()

        def idx_wait(p):
            pltpu.make_async_copy(
                src_hbm.at[wid].at[pl.ds(0, CH)], isrc[p], isem[p]).wait()
            pltpu.make_async_copy(
                dst_hbm.at[wid].at[pl.ds(0, CH)], idst[p], isem[p]).wait()

        idx_start(0, 0)
        idx_start(1, 1)
        idx_wait(0)
        g_desc(0).start()

        plsc.subcore_barrier()

        def fbody(i, carry):
            for t in range(BPB):
                g_desc(t).wait()
                s_desc(t).start(add=True)
                if with_deg:
                    if t < 2:
                        @pl.when(i > 0)
                        def _():
                            d_desc(t).wait()
                    else:
                        d_desc(t - 2).wait()
                    d_desc(t).start(add=True)
                if t == 0:
                    @pl.when(i > 0)
                    def _():
                        s_desc(BPB - 1).wait()
                    g_desc(1).start()
                else:
                    s_desc(t - 1).wait()
                    if t == 2:
                        @pl.when(i > 0)
                        def _():
                            idx_start(2 * i + 1, 1)
                    if t == 6:
                        idx_wait(1)
                    if t == 9:
                        @pl.when(i < nbod - 1)
                        def _():
                            idx_start(2 * i + 2, 0)
                    if t == 12:
                        @pl.when(i < nbod - 1)
                        def _():
                            idx_wait(0)
                    if t < BPB - 1:
                        g_desc(t + 1).start()
                    else:
                        @pl.when(i < nbod - 1)
                        def _():
                            g_desc(t + 1 - BPB).start()
            return carry

        lax.fori_loop(0, nbod, fbody, 0)

        s_desc(BPB - 1).wait()
        if with_deg:
            for t in range(2):
                d_desc(t).wait()

        plsc.subcore_barrier()

        pltpu.sync_copy(acc_sh.at[pl.ds(row0, rows_per_tile)],
                        outp_hbm.at[c, pl.ds(row0, rows_per_tile)])
        if with_deg:
            pltpu.sync_copy(deg_sh.at[pl.ds(row0, rows_per_tile)],
                            outd_hbm.at[c, pl.ds(row0, rows_per_tile)])

    return pl.kernel(
        body,
        mesh=plsc.VectorSubcoreMesh(core_axis_name="c", subcore_axis_name="s"),
        out_type=(
            jax.ShapeDtypeStruct((NC, npad, F), jnp.float32),
            jax.ShapeDtypeStruct((NC, npad), jnp.float32),
        ),
        scratch_types=(
            pltpu.VMEM_SHARED((npad, F), jnp.float32),
            pltpu.VMEM_SHARED((npad,), jnp.float32),
            pltpu.VMEM((CH, EB), jnp.int32),
            pltpu.VMEM((CH, EB), jnp.int32),
            pltpu.VMEM((CH, EB), jnp.int32),
            pltpu.VMEM((CH, EB), jnp.int32),
            pltpu.VMEM((EB, F), jnp.float32),
            pltpu.VMEM((EB, F), jnp.float32),
            pltpu.VMEM((EB,), jnp.float32),
            pltpu.VMEM((rows_per_tile,), jnp.float32),
            pltpu.SemaphoreType.DMA,
            pltpu.SemaphoreType.DMA,
            pltpu.SemaphoreType.DMA,
            pltpu.SemaphoreType.DMA,
            pltpu.SemaphoreType.DMA,
            pltpu.SemaphoreType.DMA,
            pltpu.SemaphoreType.DMA,
        ),
    )


def _mid_body(p0, p1, d0, d1, b_ref, w_ref, o_ref):
    deg = d0[...] + d1[...]
    m1n = (p0[...] + p1[...]) * (1.0 / deg)
    h = jnp.maximum(m1n + b_ref[...], 0.0)
    o_ref[...] = jnp.dot(h, w_ref[...], preferred_element_type=jnp.float32)


def _mid(p0, p1, d0, d1, b1r, w2t):
    n = p0.shape[0]
    blk = 2000
    return pl.pallas_call(
        _mid_body,
        grid=(n // blk,),
        in_specs=[
            pl.BlockSpec((blk, F), lambda i: (i, 0)),
            pl.BlockSpec((blk, F), lambda i: (i, 0)),
            pl.BlockSpec((blk, 1), lambda i: (i, 0)),
            pl.BlockSpec((blk, 1), lambda i: (i, 0)),
            pl.BlockSpec((1, F), lambda i: (0, 0)),
            pl.BlockSpec((F, F), lambda i: (0, 0)),
        ],
        out_specs=pl.BlockSpec((blk, F), lambda i: (i, 0)),
        out_shape=jax.ShapeDtypeStruct((n, F), jnp.float32),
    )(p0, p1, d0, d1, b1r, w2t)


def _out_body(q0, q1, d0, d1, b_ref, o_ref):
    ncls = o_ref.shape[1]
    deg = d0[...] + d1[...]
    z = (q0[...] + q1[...]) * (1.0 / deg) + b_ref[...]
    lane = lax.broadcasted_iota(jnp.int32, z.shape, 1)
    valid = lane < ncls
    zm = jnp.where(valid, z, -jnp.inf)
    m = jnp.max(zm, axis=1, keepdims=True)
    e = jnp.where(valid, jnp.exp(z - m), 0.0)
    sz = jnp.sum(e, axis=1, keepdims=True)
    res = z - m - jnp.log(sz)
    o_ref[...] = res[:, :ncls]


def _out(q0, q1, d0, d1, b2r, ncls):
    n = q0.shape[0]
    blk = 2000
    return pl.pallas_call(
        _out_body,
        grid=(n // blk,),
        in_specs=[
            pl.BlockSpec((blk, F), lambda i: (i, 0)),
            pl.BlockSpec((blk, F), lambda i: (i, 0)),
            pl.BlockSpec((blk, 1), lambda i: (i, 0)),
            pl.BlockSpec((blk, 1), lambda i: (i, 0)),
            pl.BlockSpec((1, F), lambda i: (0, 0)),
        ],
        out_specs=pl.BlockSpec((blk, ncls), lambda i: (i, 0)),
        out_shape=jax.ShapeDtypeStruct((n, ncls), jnp.float32),
    )(q0, q1, d0, d1, b2r)


def kernel(x, edge_index, W1, b1, W2, b2):
    n = x.shape[0]
    e = edge_index.shape[1]
    ncls = W2.shape[0]

    npad = ((n + NS * EB) // (NS * EB)) * (NS * EB)

    epw = -(-e // NW)
    nb = (-(-(-(-epw // EB)) // CH)) * CH
    slot = nb * EB
    flat_pad = NW * epw - e
    spare = npad - n
    padcols = slot - epw
    srcw = jnp.concatenate(
        [edge_index[0], jnp.full((flat_pad,), n, jnp.int32)]).reshape(NW, epw)
    dstw = jnp.concatenate(
        [edge_index[1], jnp.zeros((flat_pad,), jnp.int32)]).reshape(NW, epw)
    dump = jnp.broadcast_to(
        n + (jnp.arange(padcols, dtype=jnp.int32) % spare), (NW, padcols))
    zpad = jnp.zeros((NW, padcols), jnp.int32)
    src3 = jnp.concatenate([srcw, dump], axis=1).reshape(NW, nb, EB)
    dst3 = jnp.concatenate([dstw, zpad], axis=1).reshape(NW, nb, EB)

    w1t = W1.T
    b1r = b1.reshape(1, F)
    w2t = jnp.zeros((F, F), jnp.float32).at[:, :ncls].set(W2.T)
    b2r = jnp.zeros((1, F), jnp.float32).at[:, :ncls].set(b2.reshape(1, ncls))

    g1 = _lin1(x, w1t)

    acc1, degp = _make_agg(npad, nb, True)(g1, src3, dst3)
    d0 = degp[0, :n, None]
    d1 = degp[1, :n, None]

    g2 = _mid(acc1[0, :n], acc1[1, :n], d0, d1, b1r, w2t)

    acc2, _ = _make_agg(npad, nb, False)(g2, src3, dst3)

    return _out(acc2[0, :n], acc2[1, :n], d0, d1, b2r, ncls)
